# baseline (speedup 1.0000x reference)
import functools

import jax
import jax.numpy as jnp
from jax import lax
from jax.experimental import pallas as pl
from jax.experimental.pallas import tpu as pltpu
from jax.experimental.pallas import tpu_sc as plsc

_N_TOKEN = 100000
_D_EMBED = 1024
_D_PROJ = 2048
_VEC_LEN = 128

_NC = 2
_NS = 16
_NW = _NC * _NS

_NCHUNK = 4
_BLOCK_M = 512


def _sc_gather(table, idx_chunk, rows_per_w):
  n_rows = idx_chunk.shape[0]
  mesh = plsc.VectorSubcoreMesh(
      core_axis_name="c", subcore_axis_name="s",
      num_cores=_NC, num_subcores=_NS)

  @functools.partial(
      pl.kernel,
      out_type=jax.ShapeDtypeStruct((n_rows, _D_EMBED), jnp.float32),
      mesh=mesh,
      scratch_types=[
          pltpu.VMEM((rows_per_w,), jnp.int32),
          pltpu.VMEM((rows_per_w, _D_EMBED), jnp.float32),
          pltpu.SemaphoreType.DMA,
      ],
  )
  def gather_kernel(table_hbm, idx_hbm, out_hbm, idx_v, rows_v, sem):
    wid = lax.axis_index("s") * _NC + lax.axis_index("c")
    base = wid * rows_per_w
    pltpu.sync_copy(idx_hbm.at[pl.ds(base, rows_per_w)], idx_v)
    pltpu.async_copy(table_hbm.at[idx_v], rows_v, sem).wait()
    pltpu.sync_copy(rows_v, out_hbm.at[pl.ds(base, rows_per_w)])

  return gather_kernel(table, idx_chunk)


def _proj_kernel(o_in_ref, g_ref, s_ref, ws_ref, p_ref, o_ref):
  del o_in_ref
  x = g_ref[...] + jnp.dot(
      s_ref[...], ws_ref[...], preferred_element_type=jnp.float32)
  acc = lax.dot_general(
      x, p_ref[...], (((1,), (1,)), ((), ())),
      preferred_element_type=jnp.float32)
  o_ref[...] = acc * (_D_PROJ ** 0.5)


def _tc_project_chunk(out_buf, g, s_chunk, ws_bf, p_f32, chunk_idx, n_tok):
  blocks_per_chunk = g.shape[0] // _BLOCK_M
  base = chunk_idx * blocks_per_chunk
  return pl.pallas_call(
      _proj_kernel,
      grid=(blocks_per_chunk,),
      in_specs=[
          pl.BlockSpec(memory_space=pl.ANY),
          pl.BlockSpec((_BLOCK_M, _D_EMBED), lambda i: (i, 0)),
          pl.BlockSpec((_BLOCK_M, _VEC_LEN), lambda i: (i, 0)),
          pl.BlockSpec((_VEC_LEN, _D_EMBED), lambda i: (0, 0)),
          pl.BlockSpec((_D_PROJ, _D_EMBED), lambda i: (0, 0)),
      ],
      out_specs=pl.BlockSpec((_BLOCK_M, _D_PROJ), lambda i: (base + i, 0)),
      out_shape=jax.ShapeDtypeStruct((n_tok, _D_PROJ), jnp.float32),
      input_output_aliases={0: 0},
  )(out_buf, g, s_chunk, ws_bf, p_f32)


def kernel(inp, status_vec, emb_weight, status_weight, proj_W):
  b, l = inp.shape
  n_tok = b * l
  chunk = n_tok // _NCHUNK
  rows_per_w = chunk // _NW

  idx_flat = inp.reshape(n_tok).astype(jnp.int32)
  s_flat = status_vec.reshape(n_tok, _VEC_LEN).astype(jnp.float32)
  ws_f32 = status_weight.astype(jnp.float32)
  p_f32 = proj_W.astype(jnp.float32)

  gathered = [
      _sc_gather(emb_weight, lax.slice(idx_flat, (k * chunk,),
                                       ((k + 1) * chunk,)), rows_per_w)
      for k in range(_NCHUNK)
  ]

  out = None
  for k in range(_NCHUNK):
    s_chunk = lax.slice(s_flat, (k * chunk, 0), ((k + 1) * chunk, _VEC_LEN))
    if out is None:
      out = _tc_project_first(gathered[k], s_chunk, ws_f32, p_f32, k, n_tok)
    else:
      out = _tc_project_chunk(out, gathered[k], s_chunk, ws_f32, p_f32,
                              k, n_tok)
  return out.reshape(b, l, _D_PROJ)


def _proj_kernel_first(g_ref, s_ref, ws_ref, p_ref, o_ref):
  x = g_ref[...] + jnp.dot(
      s_ref[...], ws_ref[...], preferred_element_type=jnp.float32)
  acc = lax.dot_general(
      x, p_ref[...], (((1,), (1,)), ((), ())),
      preferred_element_type=jnp.float32)
  o_ref[...] = acc * (_D_PROJ ** 0.5)


def _tc_project_first(g, s_chunk, ws_bf, p_f32, chunk_idx, n_tok):
  blocks_per_chunk = g.shape[0] // _BLOCK_M
  base = chunk_idx * blocks_per_chunk
  return pl.pallas_call(
      _proj_kernel_first,
      grid=(blocks_per_chunk,),
      in_specs=[
          pl.BlockSpec((_BLOCK_M, _D_EMBED), lambda i: (i, 0)),
          pl.BlockSpec((_BLOCK_M, _VEC_LEN), lambda i: (i, 0)),
          pl.BlockSpec((_VEC_LEN, _D_EMBED), lambda i: (0, 0)),
          pl.BlockSpec((_D_PROJ, _D_EMBED), lambda i: (0, 0)),
      ],
      out_specs=pl.BlockSpec((_BLOCK_M, _D_PROJ), lambda i: (base + i, 0)),
      out_shape=jax.ShapeDtypeStruct((n_tok, _D_PROJ), jnp.float32),
  )(g, s_chunk, ws_bf, p_f32)


# speedup vs baseline: 1.0599x; 1.0599x over previous
import functools

import jax
import jax.numpy as jnp
from jax import lax
from jax.experimental import pallas as pl
from jax.experimental.pallas import tpu as pltpu
from jax.experimental.pallas import tpu_sc as plsc

_N_TOKEN = 100000
_D_EMBED = 1024
_D_PROJ = 2048
_VEC_LEN = 128

_NC = 2
_NS = 16
_NW = _NC * _NS

_NCHUNK = 4
_BLOCK_M = 512
_HALF = 32


def _sc_gather(table, idx_full, chunk_base, chunk_rows):
  rows_per_w = chunk_rows // _NW
  assert rows_per_w == 2 * _HALF
  mesh = plsc.VectorSubcoreMesh(
      core_axis_name="c", subcore_axis_name="s",
      num_cores=_NC, num_subcores=_NS)

  @functools.partial(
      pl.kernel,
      out_type=jax.ShapeDtypeStruct((chunk_rows, _D_EMBED), jnp.float32),
      mesh=mesh,
      scratch_types=[
          pltpu.VMEM((_HALF,), jnp.int32),
          pltpu.VMEM((_HALF,), jnp.int32),
          pltpu.VMEM((_HALF, _D_EMBED), jnp.float32),
          pltpu.VMEM((_HALF, _D_EMBED), jnp.float32),
          pltpu.SemaphoreType.DMA,
          pltpu.SemaphoreType.DMA,
          pltpu.SemaphoreType.DMA,
          pltpu.SemaphoreType.DMA,
      ],
  )
  def gather_kernel(table_hbm, idx_hbm, out_hbm,
                    idx_v0, idx_v1, rows_v0, rows_v1,
                    gsem0, gsem1, wsem0, wsem1):
    wid = lax.axis_index("s") * _NC + lax.axis_index("c")
    base = wid * (2 * _HALF)
    src0 = chunk_base + base
    src1 = chunk_base + base + _HALF
    pltpu.sync_copy(idx_hbm.at[pl.ds(src0, _HALF)], idx_v0)
    pltpu.sync_copy(idx_hbm.at[pl.ds(src1, _HALF)], idx_v1)
    g0 = pltpu.async_copy(table_hbm.at[idx_v0], rows_v0, gsem0)
    g1 = pltpu.async_copy(table_hbm.at[idx_v1], rows_v1, gsem1)
    g0.wait()
    w0 = pltpu.async_copy(rows_v0, out_hbm.at[pl.ds(base, _HALF)], wsem0)
    g1.wait()
    w1 = pltpu.async_copy(rows_v1, out_hbm.at[pl.ds(base + _HALF, _HALF)],
                          wsem1)
    w0.wait()
    w1.wait()

  return gather_kernel(table, idx_full)


def _proj_body(g_ref, s_ref, ws_ref, p_ref, o_ref):
  x = g_ref[...] + jnp.dot(
      s_ref[...], ws_ref[...], preferred_element_type=jnp.float32)
  acc = lax.dot_general(
      x.astype(jnp.bfloat16), p_ref[...], (((1,), (1,)), ((), ())),
      preferred_element_type=jnp.float32)
  o_ref[...] = acc * (_D_PROJ ** 0.5)


def _proj_kernel_first(g_ref, s_ref, ws_ref, p_ref, o_ref):
  _proj_body(g_ref, s_ref, ws_ref, p_ref, o_ref)


def _proj_kernel_next(o_in_ref, g_ref, s_ref, ws_ref, p_ref, o_ref):
  del o_in_ref
  _proj_body(g_ref, s_ref, ws_ref, p_ref, o_ref)


def _tc_project_chunk(out_buf, g, s_full, ws, p_bf, chunk_idx, n_tok):
  blocks_per_chunk = g.shape[0] // _BLOCK_M
  base = chunk_idx * blocks_per_chunk
  specs = [
      pl.BlockSpec((_BLOCK_M, _D_EMBED), lambda i: (i, 0)),
      pl.BlockSpec((_BLOCK_M, _VEC_LEN), lambda i: (base + i, 0)),
      pl.BlockSpec((_VEC_LEN, _D_EMBED), lambda i: (0, 0)),
      pl.BlockSpec((_D_PROJ, _D_EMBED), lambda i: (0, 0)),
  ]
  out_spec = pl.BlockSpec((_BLOCK_M, _D_PROJ), lambda i: (base + i, 0))
  out_shape = jax.ShapeDtypeStruct((n_tok, _D_PROJ), jnp.float32)
  if out_buf is None:
    return pl.pallas_call(
        _proj_kernel_first,
        grid=(blocks_per_chunk,),
        in_specs=specs,
        out_specs=out_spec,
        out_shape=out_shape,
    )(g, s_full, ws, p_bf)
  return pl.pallas_call(
      _proj_kernel_next,
      grid=(blocks_per_chunk,),
      in_specs=[pl.BlockSpec(memory_space=pl.ANY)] + specs,
      out_specs=out_spec,
      out_shape=out_shape,
      input_output_aliases={0: 0},
  )(out_buf, g, s_full, ws, p_bf)


def kernel(inp, status_vec, emb_weight, status_weight, proj_W):
  b, l = inp.shape
  n_tok = b * l
  chunk = n_tok // _NCHUNK

  p_bf = proj_W.astype(jnp.bfloat16)
  idx_flat = inp.reshape(n_tok).astype(jnp.int32)
  s_flat = status_vec.reshape(n_tok, _VEC_LEN).astype(jnp.float32)
  ws_f32 = status_weight.astype(jnp.float32)

  gathered = [
      _sc_gather(emb_weight, idx_flat, k * chunk, chunk)
      for k in range(_NCHUNK)
  ]

  out = None
  for k in range(_NCHUNK):
    out = _tc_project_chunk(out, gathered[k], s_flat, ws_f32, p_bf, k, n_tok)
  return out.reshape(b, l, _D_PROJ)


# speedup vs baseline: 1.0969x; 1.0350x over previous
import functools

import jax
import jax.numpy as jnp
from jax import lax
from jax.experimental import pallas as pl
from jax.experimental.pallas import tpu as pltpu
from jax.experimental.pallas import tpu_sc as plsc

_N_TOKEN = 100000
_D_EMBED = 1024
_D_PROJ = 2048
_VEC_LEN = 128

_NC = 2
_NS = 16
_NW = _NC * _NS

_BLOCK_M = 1024
_HALF = 32


def _sc_gather(table, idx_full, chunk_base, chunk_rows):
  rows_per_w = chunk_rows // _NW
  n_half = rows_per_w // _HALF
  mesh = plsc.VectorSubcoreMesh(
      core_axis_name="c", subcore_axis_name="s",
      num_cores=_NC, num_subcores=_NS)

  @functools.partial(
      pl.kernel,
      out_type=jax.ShapeDtypeStruct((chunk_rows, _D_EMBED), jnp.float32),
      mesh=mesh,
      scratch_types=[
          pltpu.VMEM((_HALF,), jnp.int32),
          pltpu.VMEM((_HALF,), jnp.int32),
          pltpu.VMEM((_HALF, _D_EMBED), jnp.float32),
          pltpu.VMEM((_HALF, _D_EMBED), jnp.float32),
          pltpu.SemaphoreType.DMA,
          pltpu.SemaphoreType.DMA,
          pltpu.SemaphoreType.DMA,
          pltpu.SemaphoreType.DMA,
      ],
  )
  def gather_kernel(table_hbm, idx_hbm, out_hbm,
                    idx_v0, idx_v1, rows_v0, rows_v1,
                    gsem0, gsem1, wsem0, wsem1):
    wid = lax.axis_index("s") * _NC + lax.axis_index("c")
    base = wid * rows_per_w

    def step(h, carry):
      off = base + 2 * h * _HALF
      src = chunk_base + off
      pltpu.sync_copy(idx_hbm.at[pl.ds(src, _HALF)], idx_v0)
      pltpu.sync_copy(idx_hbm.at[pl.ds(src + _HALF, _HALF)], idx_v1)
      g0 = pltpu.async_copy(table_hbm.at[idx_v0], rows_v0, gsem0)
      g1 = pltpu.async_copy(table_hbm.at[idx_v1], rows_v1, gsem1)
      g0.wait()
      w0 = pltpu.async_copy(rows_v0, out_hbm.at[pl.ds(off, _HALF)], wsem0)
      g1.wait()
      w1 = pltpu.async_copy(rows_v1, out_hbm.at[pl.ds(off + _HALF, _HALF)],
                            wsem1)
      w0.wait()
      w1.wait()
      return carry

    lax.fori_loop(0, n_half // 2, step, 0)

  return gather_kernel(table, idx_full)


def _proj_kernel(g_ref, s_ref, ws_ref, p_ref, o_ref):
  x = g_ref[...] + jnp.dot(
      s_ref[...], ws_ref[...], preferred_element_type=jnp.float32)
  acc = lax.dot_general(
      x, p_ref[...], (((1,), (1,)), ((), ())),
      preferred_element_type=jnp.float32)
  o_ref[...] = acc * (_D_PROJ ** 0.5)


def _tc_project(g, s_flat, ws, p, n_tok):
  return pl.pallas_call(
      _proj_kernel,
      grid=(n_tok // _BLOCK_M,),
      in_specs=[
          pl.BlockSpec((_BLOCK_M, _D_EMBED), lambda i: (i, 0)),
          pl.BlockSpec((_BLOCK_M, _VEC_LEN), lambda i: (i, 0)),
          pl.BlockSpec((_VEC_LEN, _D_EMBED), lambda i: (0, 0)),
          pl.BlockSpec((_D_PROJ, _D_EMBED), lambda i: (0, 0)),
      ],
      out_specs=pl.BlockSpec((_BLOCK_M, _D_PROJ), lambda i: (i, 0)),
      out_shape=jax.ShapeDtypeStruct((n_tok, _D_PROJ), jnp.float32),
  )(g, s_flat, ws, p)


def kernel(inp, status_vec, emb_weight, status_weight, proj_W):
  b, l = inp.shape
  n_tok = b * l

  idx_flat = inp.reshape(n_tok).astype(jnp.int32)
  s_flat = status_vec.reshape(n_tok, _VEC_LEN).astype(jnp.float32)
  ws_f32 = status_weight.astype(jnp.float32)
  p_f32 = proj_W.astype(jnp.float32)

  g = _sc_gather(emb_weight, idx_flat, 0, n_tok)
  out = _tc_project(g, s_flat, ws_f32, p_f32, n_tok)
  return out.reshape(b, l, _D_PROJ)


# speedup vs baseline: 1.1321x; 1.0321x over previous
import functools

import jax
import jax.numpy as jnp
from jax import lax
from jax.experimental import pallas as pl
from jax.experimental.pallas import tpu as pltpu
from jax.experimental.pallas import tpu_sc as plsc

_N_TOKEN = 100000
_D_EMBED = 1024
_D_PROJ = 2048
_VEC_LEN = 128

_NC = 2
_NS = 16
_NW = _NC * _NS

_CHUNKS = (2048, 3072, 3072)
_BLOCK_M = 1024


def _sc_gather(table, idx_full, chunk_base, chunk_rows):
  rows_per_w = chunk_rows // _NW
  half = rows_per_w // 2
  mesh = plsc.VectorSubcoreMesh(
      core_axis_name="c", subcore_axis_name="s",
      num_cores=_NC, num_subcores=_NS)

  @functools.partial(
      pl.kernel,
      out_type=jax.ShapeDtypeStruct((chunk_rows, _D_EMBED), jnp.float32),
      mesh=mesh,
      scratch_types=[
          pltpu.VMEM((half,), jnp.int32),
          pltpu.VMEM((half,), jnp.int32),
          pltpu.VMEM((half, _D_EMBED), jnp.float32),
          pltpu.VMEM((half, _D_EMBED), jnp.float32),
          pltpu.SemaphoreType.DMA,
          pltpu.SemaphoreType.DMA,
          pltpu.SemaphoreType.DMA,
          pltpu.SemaphoreType.DMA,
      ],
  )
  def gather_kernel(table_hbm, idx_hbm, out_hbm,
                    idx_v0, idx_v1, rows_v0, rows_v1,
                    gsem0, gsem1, wsem0, wsem1):
    wid = lax.axis_index("s") * _NC + lax.axis_index("c")
    base = wid * rows_per_w
    src = chunk_base + base
    pltpu.sync_copy(idx_hbm.at[pl.ds(src, half)], idx_v0)
    pltpu.sync_copy(idx_hbm.at[pl.ds(src + half, half)], idx_v1)
    g0 = pltpu.async_copy(table_hbm.at[idx_v0], rows_v0, gsem0)
    g1 = pltpu.async_copy(table_hbm.at[idx_v1], rows_v1, gsem1)
    g0.wait()
    w0 = pltpu.async_copy(rows_v0, out_hbm.at[pl.ds(base, half)], wsem0)
    g1.wait()
    w1 = pltpu.async_copy(rows_v1, out_hbm.at[pl.ds(base + half, half)],
                          wsem1)
    w0.wait()
    w1.wait()

  return gather_kernel(table, idx_full)


def _proj_body(g_ref, s_ref, ws_ref, p_ref, o_ref):
  x = g_ref[...] + jnp.dot(
      s_ref[...], ws_ref[...], preferred_element_type=jnp.float32)
  acc = lax.dot_general(
      x.astype(jnp.bfloat16), p_ref[...], (((1,), (1,)), ((), ())),
      preferred_element_type=jnp.float32)
  o_ref[...] = acc * (_D_PROJ ** 0.5)


def _proj_kernel_first(g_ref, s_ref, ws_ref, p_ref, o_ref):
  _proj_body(g_ref, s_ref, ws_ref, p_ref, o_ref)


def _proj_kernel_next(o_in_ref, g_ref, s_ref, ws_ref, p_ref, o_ref):
  del o_in_ref
  _proj_body(g_ref, s_ref, ws_ref, p_ref, o_ref)


def _tc_project_chunk(out_buf, g, s_full, ws, p_bf, block_base, n_tok):
  blocks = g.shape[0] // _BLOCK_M
  specs = [
      pl.BlockSpec((_BLOCK_M, _D_EMBED), lambda i: (i, 0)),
      pl.BlockSpec((_BLOCK_M, _VEC_LEN), lambda i: (block_base + i, 0)),
      pl.BlockSpec((_VEC_LEN, _D_EMBED), lambda i: (0, 0)),
      pl.BlockSpec((_D_PROJ, _D_EMBED), lambda i: (0, 0)),
  ]
  out_spec = pl.BlockSpec((_BLOCK_M, _D_PROJ), lambda i: (block_base + i, 0))
  out_shape = jax.ShapeDtypeStruct((n_tok, _D_PROJ), jnp.float32)
  if out_buf is None:
    return pl.pallas_call(
        _proj_kernel_first,
        grid=(blocks,),
        in_specs=specs,
        out_specs=out_spec,
        out_shape=out_shape,
    )(g, s_full, ws, p_bf)
  return pl.pallas_call(
      _proj_kernel_next,
      grid=(blocks,),
      in_specs=[pl.BlockSpec(memory_space=pl.ANY)] + specs,
      out_specs=out_spec,
      out_shape=out_shape,
      input_output_aliases={0: 0},
  )(out_buf, g, s_full, ws, p_bf)


def kernel(inp, status_vec, emb_weight, status_weight, proj_W):
  b, l = inp.shape
  n_tok = b * l
  assert sum(_CHUNKS) == n_tok

  p_bf = proj_W.astype(jnp.bfloat16)
  idx_flat = inp.reshape(n_tok).astype(jnp.int32)
  s_flat = status_vec.reshape(n_tok, _VEC_LEN).astype(jnp.float32)
  ws_f32 = status_weight.astype(jnp.float32)

  bases = [sum(_CHUNKS[:k]) for k in range(len(_CHUNKS))]
  gathered = [
      _sc_gather(emb_weight, idx_flat, bases[k], _CHUNKS[k])
      for k in range(len(_CHUNKS))
  ]

  out = None
  for k in range(len(_CHUNKS)):
    out = _tc_project_chunk(out, gathered[k], s_flat, ws_f32, p_bf,
                            bases[k] // _BLOCK_M, n_tok)
  return out.reshape(b, l, _D_PROJ)
